# baseline (speedup 1.0000x reference)
import functools
import numpy as np
import jax
from jax import lax
import jax.numpy as jnp
from jax.experimental import pallas as pl
from jax.experimental.pallas import tpu as pltpu
from jax.experimental.pallas import tpu_sc as plsc


def _route_body(C, E, x_ref, wg_ref, slot_ref, w_ref):
    N = x_ref.shape[0]
    xf = x_ref[...]
    logits = jnp.dot(xf, wg_ref[...], preferred_element_type=jnp.float32)
    m = jnp.max(logits, axis=1, keepdims=True)
    gv = 1.0 / jnp.sum(jnp.exp(logits - m), axis=1, keepdims=True)
    e_iota = lax.broadcasted_iota(jnp.int32, logits.shape, 1)
    idx = jnp.min(jnp.where(logits == m, e_iota, E), axis=1, keepdims=True)
    oh = (e_iota == idx).astype(jnp.float32)
    r = lax.broadcasted_iota(jnp.int32, (N, N), 0)
    c = lax.broadcasted_iota(jnp.int32, (N, N), 1)
    tri = (c < r).astype(jnp.float32)
    cum = jnp.dot(tri, oh, preferred_element_type=jnp.float32)
    pos = jnp.sum(cum * oh, axis=1, keepdims=True).astype(jnp.int32)
    keepm = pos < C
    slot_ref[...] = jnp.where(keepm, idx * C + pos, E * C)
    w_ref[...] = jnp.where(keepm, gv, 0.0)


def _h_body(C, slot_ref, x_ref, w1_ref, b1_ref, h_ref):
    e = pl.program_id(0)
    N = slot_ref.shape[0]
    slot_col = slot_ref[...]
    c_iota = lax.broadcasted_iota(jnp.int32, (N, C), 1)
    P = (slot_col == e * C + c_iota).astype(jnp.float32)
    dispx = lax.dot_general(
        P, x_ref[...], (((0,), (0,)), ((), ())),
        preferred_element_type=jnp.float32)
    h_ref[...] = jnp.maximum(
        jnp.dot(dispx, w1_ref[0], preferred_element_type=jnp.float32)
        + b1_ref[0], 0.0)


def _combine_body(C, EPB, NB, slot_ref, w_ref, h_ref, w2s_ref, b2_ref,
                  out_ref, s_acc):
    i = pl.program_id(0)
    N = slot_ref.shape[0]
    S = EPB * C
    slot_col = slot_ref[...]
    r_iota = lax.broadcasted_iota(jnp.int32, (N, S), 1)
    P = (slot_col == i * S + r_iota).astype(jnp.float32)
    rq = lax.broadcasted_iota(jnp.int32, (S, EPB), 0) // C
    jq = lax.broadcasted_iota(jnp.int32, (S, EPB), 1)
    Q = (rq == jq).astype(jnp.float32)
    w2s_blk = jnp.sum(w2s_ref[...], axis=2)
    w2s_exp = jnp.dot(Q, w2s_blk,
                      preferred_element_type=jnp.float32)
    val = jnp.sum(h_ref[...] * w2s_exp, axis=1, keepdims=True)
    b2s = jnp.sum(b2_ref[:, 0, :], axis=1, keepdims=True)
    val = val + jnp.dot(Q, b2s, preferred_element_type=jnp.float32)
    contrib = jnp.dot(P, val, preferred_element_type=jnp.float32)

    @pl.when(i == 0)
    def _():
        s_acc[...] = contrib

    @pl.when(i > 0)
    def _():
        s_acc[...] = s_acc[...] + contrib

    @pl.when(i == NB - 1)
    def _():
        s = s_acc[...] * w_ref[...]
        mx = jnp.max(s, axis=0, keepdims=True)
        lse = jnp.log(jnp.sum(jnp.exp(s - mx), axis=0, keepdims=True)) + mx
        out_ref[...] = s - lse


def _w2s_sc(R, D):
    info = plsc.get_sparse_core_info()
    nc, ns = info.num_cores, info.num_subcores
    nw = nc * ns
    rows_w = R // nw
    RC = 32
    nch = rows_w // RC
    npair = nch // 2
    nv = D // 16
    mesh = plsc.VectorSubcoreMesh(core_axis_name="c", subcore_axis_name="s")

    @functools.partial(
        pl.kernel, mesh=mesh,
        out_type=jax.ShapeDtypeStruct((R * 16,), jnp.float32),
        scratch_types=[
            pltpu.VMEM((RC, D), jnp.float32),
            pltpu.VMEM((RC, D), jnp.float32),
            pltpu.VMEM((rows_w * 16,), jnp.float32),
            pltpu.SemaphoreType.DMA,
            pltpu.SemaphoreType.DMA,
        ],
    )
    def k(w2_hbm, out_hbm, buf0, buf1, pbuf, sem0, sem1):
        wid = lax.axis_index("s") * nc + lax.axis_index("c")
        base = wid * rows_w
        pltpu.async_copy(w2_hbm.at[pl.ds(base, RC), :], buf0, sem0)
        pltpu.async_copy(w2_hbm.at[pl.ds(base + RC, RC), :], buf1, sem1)

        def reduce_chunk(buf, g):
            def row_body(rr, _):
                accs = [buf[rr, pl.ds(16 * a, 16)] for a in range(4)]
                for j in range(4, nv):
                    a = j % 4
                    accs[a] = accs[a] + buf[rr, pl.ds(16 * j, 16)]
                pbuf[pl.ds((g * RC + rr) * 16, 16)] = \
                    (accs[0] + accs[1]) + (accs[2] + accs[3])
                return 0
            lax.fori_loop(0, RC, row_body, 0)

        def pair_body(p, _):
            g0 = 2 * p
            pltpu.make_async_copy(
                w2_hbm.at[pl.ds(base, RC), :], buf0, sem0).wait()
            reduce_chunk(buf0, g0)

            @pl.when(p + 1 < npair)
            def _():
                pltpu.async_copy(
                    w2_hbm.at[pl.ds(base + (g0 + 2) * RC, RC), :],
                    buf0, sem0)

            pltpu.make_async_copy(
                w2_hbm.at[pl.ds(base, RC), :], buf1, sem1).wait()
            reduce_chunk(buf1, g0 + 1)

            @pl.when(p + 1 < npair)
            def _():
                pltpu.async_copy(
                    w2_hbm.at[pl.ds(base + (g0 + 3) * RC, RC), :],
                    buf1, sem1)
            return 0

        lax.fori_loop(0, npair, pair_body, 0)
        pltpu.sync_copy(pbuf, out_hbm.at[pl.ds(base * 16, rows_w * 16)])

    return k


def kernel(x, Wg, W1, b1, W2, b2):
    B_, T_, D_ = x.shape
    N = B_ * T_
    E_ = Wg.shape[1]
    H_ = W1.shape[2]
    C = int(np.ceil(N * 1.25 / E_))
    EPB = 8
    NB = E_ // EPB
    xf = x.reshape(N, D_)

    w2p = _w2s_sc(E_ * H_, D_)(W2.reshape(E_ * H_, D_))
    w2p = w2p.reshape(E_, H_, 16)

    slot, w = pl.pallas_call(
        functools.partial(_route_body, C, E_),
        out_shape=[jax.ShapeDtypeStruct((N, 1), jnp.int32),
                   jax.ShapeDtypeStruct((N, 1), jnp.float32)],
    )(xf, Wg)

    h_all = pl.pallas_call(
        functools.partial(_h_body, C),
        grid=(E_,),
        in_specs=[
            pl.BlockSpec((N, 1), lambda e: (0, 0)),
            pl.BlockSpec((N, D_), lambda e: (0, 0)),
            pl.BlockSpec((1, D_, H_), lambda e: (e, 0, 0)),
            pl.BlockSpec((1, 1, H_), lambda e: (e, 0, 0)),
        ],
        out_specs=pl.BlockSpec((C, H_), lambda e: (e, 0)),
        out_shape=jax.ShapeDtypeStruct((E_ * C, H_), jnp.float32),
    )(slot, xf, W1, b1.reshape(E_, 1, H_))

    out = pl.pallas_call(
        functools.partial(_combine_body, C, EPB, NB),
        grid=(NB,),
        in_specs=[
            pl.BlockSpec((N, 1), lambda i: (0, 0)),
            pl.BlockSpec((N, 1), lambda i: (0, 0)),
            pl.BlockSpec((EPB * C, H_), lambda i: (i, 0)),
            pl.BlockSpec((EPB, H_, 16), lambda i: (i, 0, 0)),
            pl.BlockSpec((EPB, 1, D_), lambda i: (i, 0, 0)),
        ],
        out_specs=pl.BlockSpec((N, 1), lambda i: (0, 0)),
        out_shape=jax.ShapeDtypeStruct((N, 1), jnp.float32),
        scratch_shapes=[pltpu.VMEM((N, 1), jnp.float32)],
    )(slot, w, h_all, w2p, b2.reshape(E_, 1, D_))
    return out.reshape(B_, T_)


# speedup vs baseline: 1.4417x; 1.4417x over previous
import functools
import numpy as np
import jax
from jax import lax
import jax.numpy as jnp
from jax.experimental import pallas as pl
from jax.experimental.pallas import tpu as pltpu


def _body(C, E, NBK, x_ref, wg_ref, w1_ref, b1_ref, w2_ref, b2_ref,
          out_ref, slot_s, w_s, s_acc):
    e = pl.program_id(0)
    N = x_ref.shape[0]

    @pl.when(e == 0)
    def _():
        xf = x_ref[...]
        logits = jnp.dot(xf, wg_ref[...], preferred_element_type=jnp.float32)
        m = jnp.max(logits, axis=1, keepdims=True)
        gv = 1.0 / jnp.sum(jnp.exp(logits - m), axis=1, keepdims=True)
        e_iota = lax.broadcasted_iota(jnp.int32, logits.shape, 1)
        idx = jnp.min(jnp.where(logits == m, e_iota, E), axis=1, keepdims=True)
        oh = (e_iota == idx).astype(jnp.float32)
        BS = N // NBK
        r = lax.broadcasted_iota(jnp.int32, (BS, BS), 0)
        c2 = lax.broadcasted_iota(jnp.int32, (BS, BS), 1)
        tri = (c2 < r).astype(jnp.float32)
        base = jnp.zeros((1, E), jnp.float32)
        for b in range(NBK):
            sl = slice(b * BS, (b + 1) * BS)
            ohb = oh[sl, :]
            cum = jnp.dot(tri, ohb, preferred_element_type=jnp.float32) + base
            posb = jnp.sum(cum * ohb, axis=1, keepdims=True).astype(jnp.int32)
            keepb = posb < C
            slot_s[sl, :] = jnp.where(keepb, idx[sl, :] * C + posb, E * C)
            w_s[sl, :] = jnp.where(keepb, gv[sl, :], 0.0)
            base = base + jnp.sum(ohb, axis=0, keepdims=True)

    slot_col = slot_s[...]
    c_iota = lax.broadcasted_iota(jnp.int32, (N, C), 1)
    P = (slot_col == e * C + c_iota).astype(jnp.float32)
    dispx = lax.dot_general(
        P, x_ref[...], (((0,), (0,)), ((), ())),
        preferred_element_type=jnp.float32)
    h = jnp.maximum(
        jnp.dot(dispx, w1_ref[0], preferred_element_type=jnp.float32)
        + b1_ref[0], 0.0)
    w2s = jnp.sum(w2_ref[0], axis=1, keepdims=True)
    val = jnp.dot(h, w2s, preferred_element_type=jnp.float32) \
        + jnp.sum(b2_ref[0])
    contrib = jnp.dot(P, val, preferred_element_type=jnp.float32) \
        * w_s[...]

    @pl.when(e == 0)
    def _():
        s_acc[...] = contrib

    @pl.when(e > 0)
    def _():
        s_acc[...] = s_acc[...] + contrib

    @pl.when(e == E - 1)
    def _():
        s = s_acc[...]
        mx = jnp.max(s, axis=0, keepdims=True)
        lse = jnp.log(jnp.sum(jnp.exp(s - mx), axis=0, keepdims=True)) + mx
        out_ref[...] = s - lse


def kernel(x, Wg, W1, b1, W2, b2):
    B_, T_, D_ = x.shape
    N = B_ * T_
    E_ = Wg.shape[1]
    H_ = W1.shape[2]
    C = int(np.ceil(N * 1.25 / E_))
    xf = x.reshape(N, D_)

    out = pl.pallas_call(
        functools.partial(_body, C, E_, 16),
        grid=(E_,),
        in_specs=[
            pl.BlockSpec((N, D_), lambda e: (0, 0)),
            pl.BlockSpec((D_, E_), lambda e: (0, 0)),
            pl.BlockSpec((1, D_, H_), lambda e: (e, 0, 0)),
            pl.BlockSpec((1, 1, H_), lambda e: (e, 0, 0)),
            pl.BlockSpec((1, H_, D_), lambda e: (e, 0, 0)),
            pl.BlockSpec((1, 1, D_), lambda e: (e, 0, 0)),
        ],
        out_specs=pl.BlockSpec((N, 1), lambda e: (0, 0)),
        out_shape=jax.ShapeDtypeStruct((N, 1), jnp.float32),
        scratch_shapes=[pltpu.VMEM((N, 1), jnp.int32),
                        pltpu.VMEM((N, 1), jnp.float32),
                        pltpu.VMEM((N, 1), jnp.float32)],
    )(xf, Wg, W1, b1.reshape(E_, 1, H_), W2, b2.reshape(E_, 1, D_))
    return out.reshape(B_, T_)
